# baseline (speedup 1.0000x reference)
import functools

import jax
import jax.numpy as jnp
from jax import lax
from jax.experimental import pallas as pl
from jax.experimental.pallas import tpu as pltpu
from jax.experimental.pallas import tpu_sc as plsc

_SINK = 4
_RECENT = 64
_KEEP_RATIO = 0.5
_ALPHA = 0.2

_L = 16

_NB1 = 1056
_NB2 = 1024
_NB3 = 1024
_P1 = _NB1 + 1
_P2 = _NB2 + 1
_P3 = _NB3 + 1


def _pad128(n):
    return (n + 127) // 128 * 128


_H1SZ = _pad128(_L * _P1)
_H2SZ = _pad128(_L * _P2)
_H3SZ = _pad128(_L * _P3)


def _floor_div(a, b):
    q = jax.lax.div(a, jnp.int32(b))
    r = a - q * b
    return q - jnp.where(r < 0, jnp.int32(1), jnp.int32(0))


def _row_sum_mxu(x, n_out=8):
    ones = jnp.ones((x.shape[1], n_out), x.dtype)
    out = jax.lax.dot_general(
        x, ones,
        dimension_numbers=(((1,), (0,)), ((), ())),
        preferred_element_type=jnp.float32,
    )
    return out[:, :1]


def _dense_body(w_ref, bits_ref, bud_ref, *, sink, recent, total_mid_budget,
                min_budget):
    _, H, L_q, L_kv = w_ref.shape
    eps = jnp.float32(1e-8)
    w = w_ref[0]

    scores = jnp.sum(w, axis=1)
    w2 = w.reshape(H * L_q, L_kv)
    ent_elem = w2 * jnp.log(w2 + eps)
    ent_hq = _row_sum_mxu(ent_elem)
    ent = jnp.sum(ent_hq.reshape(H, L_q), axis=1, keepdims=True)

    middle_len = L_kv - sink - recent
    head_entropy = -ent / jnp.float32(L_q)
    denom = jnp.sum(head_entropy) + eps
    alloc = head_entropy / denom
    budgets = jnp.round(alloc * jnp.float32(total_mid_budget)).astype(jnp.int32)
    budgets = jnp.maximum(budgets, jnp.int32(min_budget))
    current_total = jnp.sum(budgets)
    diff = jnp.int32(total_mid_budget) - current_total
    per_head_adj = _floor_div(diff, H)
    budgets = budgets + per_head_adj
    r = diff - per_head_adj * H
    idx_h = jax.lax.broadcasted_iota(jnp.int32, (H, 1), 0)
    budgets = budgets + jnp.where(idx_h < r, jnp.int32(1), jnp.int32(0))
    budgets = jnp.clip(budgets, jnp.int32(1), jnp.int32(middle_len))

    bits = jax.lax.bitcast_convert_type(scores, jnp.int32)
    col = jax.lax.broadcasted_iota(jnp.int32, (H, L_kv), 1)
    mid_mask = (col >= sink) & (col < L_kv - recent)
    bits_ref[0] = jnp.where(mid_mask, bits, jnp.int32(-1))
    bud_ref[0] = budgets.reshape(1, H)


def _splat(vec, k):
    lane = lax.broadcasted_iota(jnp.int32, (_L,), 0)
    masked = jnp.where(lane == k, vec, jnp.int32(-0x80000000))
    return jnp.broadcast_to(jnp.max(masked), (_L,))


def _select_body(bits_hbm, bud_hbm, out_hbm, buf, budv, hist1, hist2, hist3,
                 sem, *, sink, recent, l_kv):
    U = 8
    n_vec = l_kv // _L
    wid = lax.axis_index("s") * 2 + lax.axis_index("c")
    base = wid * _L
    lane = lax.broadcasted_iota(jnp.int32, (_L,), 0)
    ones16 = jnp.ones((_L,), jnp.int32)
    zeros16 = jnp.zeros((_L,), jnp.int32)

    pltpu.sync_copy(bud_hbm.at[pl.ds(base, _L)], budv)
    bud = budv[...]

    copies = [
        pltpu.async_copy(bits_hbm.at[base + k],
                         buf.at[pl.ds(k * l_kv, l_kv)], sem)
        for k in range(_L)
    ]
    for c in copies:
        c.wait()

    def _zero(hist, n):
        def _z(tt, c):
            for u in range(U):
                hist[pl.ds((tt * U + u) * _L, _L)] = zeros16
            return c
        lax.fori_loop(0, n // (U * _L), _z, 0)

    _zero(hist1, _H1SZ)
    _zero(hist2, _H2SZ)
    _zero(hist3, _H3SZ)

    for k in range(_L):
        def _h1(jj, c, k=k):
            for u in range(U):
                j = jj * U + u
                v = buf[pl.ds(k * l_kv + j * _L, _L)]
                valid = v >= 0
                b1 = jnp.clip(jnp.right_shift(v, 20), 0, _NB1 - 1)
                cnt, last = plsc.scan_count(b1, valid)
                plsc.addupdate_scatter(hist1, [k * _P1 + b1], cnt, mask=last)
            return c

        lax.fori_loop(0, n_vec // U, _h1, 0)

    def _scan(hist, pitch, nb, target):
        def _s(tt, carry):
            cum, beta, above, found = carry
            for u in range(U):
                b = nb - 1 - (tt * U + u)
                h = plsc.load_gather(hist, [lane * pitch + b])
                cum2 = cum + h
                sel = (found == 0) & (cum2 >= target)
                bv = jnp.broadcast_to(b, (_L,))
                beta = jnp.where(sel, bv, beta)
                above = jnp.where(sel, cum, above)
                found = jnp.where(sel, ones16, found)
                cum = cum2
            return cum, beta, above, found

        _, beta, above, _ = lax.fori_loop(
            0, nb // U, _s, (zeros16, zeros16, zeros16, zeros16))
        return beta, above

    beta1, above1 = _scan(hist1, _P1, _NB1, bud)
    m1 = bud - above1

    for k in range(_L):
        b1k = _splat(beta1, k)

        def _h2(jj, c, k=k, b1k=b1k):
            for u in range(U):
                j = jj * U + u
                v = buf[pl.ds(k * l_kv + j * _L, _L)]
                valid = (v >= 0) & (jnp.right_shift(v, 20) == b1k)
                b2 = jnp.bitwise_and(jnp.right_shift(v, 10), _NB2 - 1)
                cnt, last = plsc.scan_count(b2, valid)
                plsc.addupdate_scatter(hist2, [k * _P2 + b2], cnt, mask=last)
            return c

        lax.fori_loop(0, n_vec // U, _h2, 0)

    beta2, above2 = _scan(hist2, _P2, _NB2, m1)
    m2 = m1 - above2

    pref = jnp.bitwise_or(jnp.left_shift(beta1, 10), beta2)
    for k in range(_L):
        prefk = _splat(pref, k)

        def _h3(jj, c, k=k, prefk=prefk):
            for u in range(U):
                j = jj * U + u
                v = buf[pl.ds(k * l_kv + j * _L, _L)]
                valid = (v >= 0) & (jnp.right_shift(v, 10) == prefk)
                b3 = jnp.bitwise_and(v, _NB3 - 1)
                cnt, last = plsc.scan_count(b3, valid)
                plsc.addupdate_scatter(hist3, [k * _P3 + b3], cnt, mask=last)
            return c

        lax.fori_loop(0, n_vec // U, _h3, 0)

    beta3, above3 = _scan(hist3, _P3, _NB3, m2)
    m_tie = m2 - above3

    thr = jnp.bitwise_or(jnp.left_shift(pref, 10), beta3)

    out_copies = []
    for k in range(_L):
        thrk = _splat(thr, k)
        mtk = _splat(m_tie, k)

        def _fin(jj, carry, k=k, thrk=thrk, mtk=mtk):
            for u in range(U):
                j = jj * U + u
                v = buf[pl.ds(k * l_kv + j * _L, _L)]
                tie = v == thrk
                prefix = plsc.cumsum(tie.astype(jnp.int32))
                kept = tie & ((prefix + carry) <= mtk)
                pos = jnp.broadcast_to(j * _L, (_L,)) + lane
                prot = (pos < sink) | (pos >= l_kv - recent)
                keep = (v > thrk) | kept | prot
                buf[pl.ds(k * l_kv + j * _L, _L)] = keep.astype(jnp.int32)
                carry = carry + jnp.broadcast_to(jnp.max(prefix), (_L,))
            return carry

        lax.fori_loop(0, n_vec // U, _fin, zeros16)
        out_copies.append(
            pltpu.async_copy(buf.at[pl.ds(k * l_kv, l_kv)],
                             out_hbm.at[base + k], sem))

    for c in out_copies:
        c.wait()


def kernel(attn_weights):
    B, H, L_q, L_kv = attn_weights.shape
    sink, recent = _SINK, _RECENT
    n_protected = min(sink + recent, L_kv)
    middle_len = L_kv - n_protected
    if middle_len <= 0:
        return jnp.ones((B, H, L_kv), dtype=bool)
    total_keep = int(L_kv * _KEEP_RATIO)
    middle_budget = max(total_keep - n_protected, 0)
    total_mid_budget = middle_budget * H
    min_budget = max(int(middle_len * _KEEP_RATIO * _ALPHA), 1)

    dense = functools.partial(
        _dense_body, sink=sink, recent=recent,
        total_mid_budget=total_mid_budget, min_budget=min_budget)
    bits, budgets = pl.pallas_call(
        dense,
        grid=(B,),
        in_specs=[pl.BlockSpec((1, H, L_q, L_kv), lambda b: (b, 0, 0, 0))],
        out_specs=[
            pl.BlockSpec((1, H, L_kv), lambda b: (b, 0, 0)),
            pl.BlockSpec((1, 1, H), lambda b: (b, 0, 0)),
        ],
        out_shape=[
            jax.ShapeDtypeStruct((B, H, L_kv), jnp.int32),
            jax.ShapeDtypeStruct((B, 1, H), jnp.int32),
        ],
    )(attn_weights)

    bits2 = bits.reshape(B * H, L_kv)
    bud2 = budgets.reshape(B * H)

    mesh = plsc.VectorSubcoreMesh(core_axis_name="c", subcore_axis_name="s")
    select = functools.partial(
        _select_body, sink=sink, recent=recent, l_kv=L_kv)
    maski = pl.kernel(
        select,
        mesh=mesh,
        compiler_params=pltpu.CompilerParams(needs_layout_passes=False),
        out_type=jax.ShapeDtypeStruct((B * H, L_kv), jnp.int32),
        scratch_types=[
            pltpu.VMEM((_L * L_kv,), jnp.int32),
            pltpu.VMEM((_L,), jnp.int32),
            pltpu.VMEM((_H1SZ,), jnp.int32),
            pltpu.VMEM((_H2SZ,), jnp.int32),
            pltpu.VMEM((_H3SZ,), jnp.int32),
            pltpu.SemaphoreType.DMA,
        ],
    )(bits2, bud2)

    return maski.astype(jnp.bool_).reshape(B, H, L_kv)


# speedup vs baseline: 2.5234x; 2.5234x over previous
import functools

import jax
import jax.numpy as jnp
from jax import lax
from jax.experimental import pallas as pl
from jax.experimental.pallas import tpu as pltpu
from jax.experimental.pallas import tpu_sc as plsc

_SINK = 4
_RECENT = 64
_KEEP_RATIO = 0.5
_ALPHA = 0.2

_L = 16
_NW = 32

_NB1 = 1056
_NB2 = 1024
_NB3 = 1024


def _floor_div(a, b):
    q = jax.lax.div(a, jnp.int32(b))
    r = a - q * b
    return q - jnp.where(r < 0, jnp.int32(1), jnp.int32(0))


def _row_sum_mxu(x, n_out=8):
    ones = jnp.ones((x.shape[1], n_out), x.dtype)
    out = jax.lax.dot_general(
        x, ones,
        dimension_numbers=(((1,), (0,)), ((), ())),
        preferred_element_type=jnp.float32,
    )
    return out[:, :1]


def _dense_body(w_ref, bits_ref, bud_ref, *, sink, recent, total_mid_budget,
                min_budget):
    _, H, L_q, L_kv = w_ref.shape
    eps = jnp.float32(1e-8)
    w = w_ref[0]

    scores = jnp.sum(w, axis=1)
    w2 = w.reshape(H * L_q, L_kv)
    ent_elem = w2 * jnp.log(w2 + eps)
    ent_hq = _row_sum_mxu(ent_elem)
    ent = jnp.sum(ent_hq.reshape(H, L_q), axis=1, keepdims=True)

    middle_len = L_kv - sink - recent
    head_entropy = -ent / jnp.float32(L_q)
    denom = jnp.sum(head_entropy) + eps
    alloc = head_entropy / denom
    budgets = jnp.round(alloc * jnp.float32(total_mid_budget)).astype(jnp.int32)
    budgets = jnp.maximum(budgets, jnp.int32(min_budget))
    current_total = jnp.sum(budgets)
    diff = jnp.int32(total_mid_budget) - current_total
    per_head_adj = _floor_div(diff, H)
    budgets = budgets + per_head_adj
    r = diff - per_head_adj * H
    idx_h = jax.lax.broadcasted_iota(jnp.int32, (H, 1), 0)
    budgets = budgets + jnp.where(idx_h < r, jnp.int32(1), jnp.int32(0))
    budgets = jnp.clip(budgets, jnp.int32(1), jnp.int32(middle_len))

    bits = jax.lax.bitcast_convert_type(scores, jnp.int32)
    col = jax.lax.broadcasted_iota(jnp.int32, (H, L_kv), 1)
    mid_mask = (col >= sink) & (col < L_kv - recent)
    bits_ref[0] = jnp.where(mid_mask, bits, jnp.int32(-1))
    bud_ref[0] = budgets.reshape(1, H)


def _select_body(bits_hbm, bud_hbm, out_hbm, buf, stage, budv, hist1, hist2,
                 hist3, sem0, sem1, *, sink, recent, l_kv):
    n_vec = l_kv // _L
    wid = lax.axis_index("s") * 2 + lax.axis_index("c")
    base = wid * _L
    lane = lax.broadcasted_iota(jnp.int32, (_L,), 0)
    ones16 = jnp.ones((_L,), jnp.int32)
    zeros16 = jnp.zeros((_L,), jnp.int32)
    row_base = lane * l_kv
    lkv_mask = jnp.int32(l_kv - 1)

    pltpu.sync_copy(bud_hbm.at[pl.ds(base, _L)], budv)
    bud = budv[...]

    def tidx(i):
        return row_base + jnp.bitwise_and(i + lane, lkv_mask)

    U = 8
    sems = (sem0, sem1)

    in_copies = [
        pltpu.async_copy(bits_hbm.at[base + k],
                         stage.at[pl.ds((k % 2) * l_kv, l_kv)], sems[k % 2])
        for k in range(2)
    ]
    for k in range(_L):
        in_copies[k].wait()
        half = (k % 2) * l_kv

        @plsc.parallel_loop(0, n_vec, unroll=U)
        def _cp(j, k=k, half=half):
            v = stage[pl.ds(half + j * _L, _L)]
            off = jnp.bitwise_and(j * _L + k + lane, lkv_mask)
            plsc.store_scatter(buf, [k * l_kv + off], v)

        if k + 2 < _L:
            in_copies.append(
                pltpu.async_copy(bits_hbm.at[base + k + 2],
                                 stage.at[pl.ds((k % 2) * l_kv, l_kv)],
                                 sems[k % 2]))

    def _zero(hist, nb):
        @plsc.parallel_loop(0, nb, unroll=U)
        def _z(b):
            hist[pl.ds(b * _L, _L)] = zeros16

    _zero(hist1, _NB1)
    _zero(hist2, _NB2)
    _zero(hist3, _NB3)

    def gval(i):
        return plsc.load_gather(buf, [tidx(i)])

    @plsc.parallel_loop(0, l_kv, unroll=U)
    def _h1(i):
        v = gval(i)
        valid = v >= 0
        b1 = jnp.clip(jnp.right_shift(v, 20), 0, _NB1 - 1)
        plsc.addupdate_scatter(hist1, [b1 * _L + lane], ones16, mask=valid)

    def _scan(hist, nb, target):
        def _s(t, carry):
            cum, beta, above, found = carry
            b = nb - 1 - t
            h = hist[pl.ds(b * _L, _L)]
            cum2 = cum + h
            sel = (found == 0) & (cum2 >= target)
            bv = jnp.broadcast_to(b, (_L,))
            beta = jnp.where(sel, bv, beta)
            above = jnp.where(sel, cum, above)
            found = jnp.where(sel, ones16, found)
            return cum2, beta, above, found

        _, beta, above, _ = plsc.parallel_loop(
            0, nb, unroll=U, carry=(zeros16, zeros16, zeros16, zeros16))(_s)
        return beta, above

    beta1, above1 = _scan(hist1, _NB1, bud)
    m1 = bud - above1

    @plsc.parallel_loop(0, l_kv, unroll=U)
    def _h2(i):
        v = gval(i)
        b1 = jnp.right_shift(v, 20)
        valid = (v >= 0) & (b1 == beta1)
        b2 = jnp.bitwise_and(jnp.right_shift(v, 10), _NB2 - 1)
        plsc.addupdate_scatter(hist2, [b2 * _L + lane], ones16, mask=valid)

    beta2, above2 = _scan(hist2, _NB2, m1)
    m2 = m1 - above2

    @plsc.parallel_loop(0, l_kv, unroll=U)
    def _h3(i):
        v = gval(i)
        b1 = jnp.right_shift(v, 20)
        b2 = jnp.bitwise_and(jnp.right_shift(v, 10), _NB2 - 1)
        valid = (v >= 0) & (b1 == beta1) & (b2 == beta2)
        b3 = jnp.bitwise_and(v, _NB3 - 1)
        plsc.addupdate_scatter(hist3, [b3 * _L + lane], ones16, mask=valid)

    beta3, above3 = _scan(hist3, _NB3, m2)
    m_tie = m2 - above3

    thr = jnp.bitwise_or(
        jnp.left_shift(beta1, 20),
        jnp.bitwise_or(jnp.left_shift(beta2, 10), beta3))

    def _fin(i, tiecnt):
        v = gval(i)
        tie = v == thr
        keep = (v > thr) | (tie & (tiecnt < m_tie))
        prot = (i < sink) | (i >= l_kv - recent)
        keepi = jnp.bitwise_or(
            keep.astype(jnp.int32),
            jnp.broadcast_to(prot.astype(jnp.int32), (_L,)))
        plsc.store_scatter(buf, [tidx(i)], keepi)
        return tiecnt + tie.astype(jnp.int32)

    plsc.parallel_loop(0, l_kv, unroll=U, carry=zeros16)(_fin)

    out_copies = [None, None]
    for k in range(_L):
        if out_copies[k % 2] is not None:
            out_copies[k % 2].wait()
        half = (k % 2) * l_kv

        @plsc.parallel_loop(0, n_vec, unroll=U)
        def _ob(j, k=k, half=half):
            off = jnp.bitwise_and(j * _L + k + lane, lkv_mask)
            stage[pl.ds(half + j * _L, _L)] = plsc.load_gather(
                buf, [k * l_kv + off])

        out_copies[k % 2] = pltpu.async_copy(
            stage.at[pl.ds(half, l_kv)], out_hbm.at[base + k], sems[k % 2])

    out_copies[0].wait()
    out_copies[1].wait()


def kernel(attn_weights):
    B, H, L_q, L_kv = attn_weights.shape
    sink, recent = _SINK, _RECENT
    n_protected = min(sink + recent, L_kv)
    middle_len = L_kv - n_protected
    if middle_len <= 0:
        return jnp.ones((B, H, L_kv), dtype=bool)
    total_keep = int(L_kv * _KEEP_RATIO)
    middle_budget = max(total_keep - n_protected, 0)
    total_mid_budget = middle_budget * H
    min_budget = max(int(middle_len * _KEEP_RATIO * _ALPHA), 1)

    dense = functools.partial(
        _dense_body, sink=sink, recent=recent,
        total_mid_budget=total_mid_budget, min_budget=min_budget)
    bits, budgets = pl.pallas_call(
        dense,
        grid=(B,),
        in_specs=[pl.BlockSpec((1, H, L_q, L_kv), lambda b: (b, 0, 0, 0))],
        out_specs=[
            pl.BlockSpec((1, H, L_kv), lambda b: (b, 0, 0)),
            pl.BlockSpec((1, 1, H), lambda b: (b, 0, 0)),
        ],
        out_shape=[
            jax.ShapeDtypeStruct((B, H, L_kv), jnp.int32),
            jax.ShapeDtypeStruct((B, 1, H), jnp.int32),
        ],
    )(attn_weights)

    bits2 = bits.reshape(B * H, L_kv)
    bud2 = budgets.reshape(B * H)

    mesh = plsc.VectorSubcoreMesh(core_axis_name="c", subcore_axis_name="s")
    select = functools.partial(
        _select_body, sink=sink, recent=recent, l_kv=L_kv)
    maski = pl.kernel(
        select,
        mesh=mesh,
        compiler_params=pltpu.CompilerParams(needs_layout_passes=False),
        out_type=jax.ShapeDtypeStruct((B * H, L_kv), jnp.int32),
        scratch_types=[
            pltpu.VMEM((_L * L_kv,), jnp.int32),
            pltpu.VMEM((2 * L_kv,), jnp.int32),
            pltpu.VMEM((_L,), jnp.int32),
            pltpu.VMEM((_NB1 * _L,), jnp.int32),
            pltpu.VMEM((_NB2 * _L,), jnp.int32),
            pltpu.VMEM((_NB3 * _L,), jnp.int32),
            pltpu.SemaphoreType.DMA,
            pltpu.SemaphoreType.DMA,
        ],
    )(bits2, bud2)

    return maski.astype(jnp.bool_).reshape(B, H, L_kv)
